# baseline (speedup 1.0000x reference)
import functools

import jax
import jax.numpy as jnp
from jax import lax
from jax.experimental import pallas as pl
from jax.experimental.pallas import tpu as pltpu
from jax.experimental.pallas import tpu_sc as plsc

N = 10000
D = 128
QD = D // 4
E = 320000
NC = 2
NS = 16
NW = NC * NS
L = 16

CB = 128
CH = 160
EPT = E // NS
EPP = CH * CB
DCH = (NS * EPP) // (NW * CB)
ACC_ROWS = 10112
RPT = ACC_ROWS // NS
NPT = N // NS
DUMMY = 10016

_mesh = plsc.VectorSubcoreMesh(
    core_axis_name="c", subcore_axis_name="s", num_cores=NC, num_subcores=NS)
_sc_params = pltpu.CompilerParams(
    needs_layout_passes=False, use_tc_tiling_on_sc=False)


@functools.partial(
    pl.kernel,
    out_type=jax.ShapeDtypeStruct((NW, ACC_ROWS), jnp.float32),
    mesh=_mesh,
    compiler_params=_sc_params,
    scratch_types=[
        pltpu.VMEM((DCH, CB), jnp.int32),
        pltpu.VMEM((ACC_ROWS,), jnp.float32),
    ],
)
def _deg_kernel(dstr_hbm, deg_out, dst_v, deg_v):
    wid = lax.axis_index("s") * NC + lax.axis_index("c")
    pltpu.sync_copy(dstr_hbm.at[wid], dst_v)

    zeros16 = jnp.zeros((L,), jnp.float32)

    def zero_body(i, _):
        deg_v[pl.ds(i * L, L)] = zeros16
        return 0

    lax.fori_loop(0, ACC_ROWS // L, zero_body, 0)

    ones16 = jnp.full((L,), 1.0, jnp.float32)

    def acc_body(t, _):
        j = t >> 3
        k = t & 7
        idx = dst_v[j, pl.ds(k * L, L)]
        plsc.addupdate_scatter(deg_v, [idx], ones16)
        return 0

    lax.fori_loop(0, DCH * (CB // L), acc_body, 0)
    pltpu.sync_copy(deg_v, deg_out.at[wid])


@functools.partial(
    pl.kernel,
    out_type=jax.ShapeDtypeStruct((ACC_ROWS, D), jnp.float32),
    mesh=_mesh,
    compiler_params=_sc_params,
    scratch_types=[
        pltpu.VMEM((CH, CB), jnp.int32),
        pltpu.VMEM((CH, CB), jnp.int32),
        [pltpu.VMEM((CB, QD), jnp.float32) for _ in range(4)],
        pltpu.VMEM_SHARED((N, QD), jnp.float32),
        pltpu.VMEM_SHARED((ACC_ROWS, QD), jnp.float32),
        [pltpu.SemaphoreType.DMA for _ in range(4)],
        [pltpu.SemaphoreType.DMA for _ in range(4)],
    ],
)
def _mp_kernel(hp_hbm, srcr_hbm, dstr_hbm, agg_out,
               src_v, dst_v, bufs, hp_s, acc, gsems, ssems):
    c = lax.axis_index("c")
    s = lax.axis_index("s")

    pltpu.sync_copy(srcr_hbm.at[s], src_v)
    pltpu.sync_copy(dstr_hbm.at[s], dst_v)

    r0 = bufs[0]
    zeros16 = jnp.zeros((L,), jnp.float32)

    def zero_body(t, _):
        j = t >> 1
        k = t & 1
        r0[j, pl.ds(k * L, L)] = zeros16
        return 0

    lax.fori_loop(0, CB * (QD // L), zero_body, 0)

    for h in range(2):
        for q in range(4):
            pltpu.sync_copy(r0, acc.at[pl.ds(s * RPT + q * CB, CB)])
        pltpu.sync_copy(r0.at[pl.ds(0, RPT - 4 * CB)],
                        acc.at[pl.ds(s * RPT + 4 * CB, RPT - 4 * CB)])
        qoff = (2 * c + h) * QD
        pltpu.sync_copy(hp_hbm.at[pl.ds(s * NPT, NPT), pl.ds(qoff, QD)],
                        hp_s.at[pl.ds(s * NPT, NPT)])
        plsc.subcore_barrier()

        for b in range(4):
            pltpu.async_copy(hp_s.at[src_v.at[b]], bufs[b], gsems[b])

        def body(t, _):
            j = 4 * t
            scat = []
            for b in range(4):
                pltpu.make_async_copy(
                    hp_s.at[src_v.at[j + b]], bufs[b], gsems[b]).wait()
                scat.append(pltpu.async_copy(
                    bufs[b], acc.at[dst_v.at[j + b]], ssems[b], add=True))
            for b in range(4):
                scat[b].wait()
                n = jnp.minimum(j + 4 + b, CH - 4 + b)
                pltpu.async_copy(hp_s.at[src_v.at[n]], bufs[b], gsems[b])
            return 0

        lax.fori_loop(0, CH // 4, body, 0)

        for b in range(4):
            pltpu.make_async_copy(
                hp_s.at[src_v.at[CH - 4 + b]], bufs[b], gsems[b]).wait()

        plsc.subcore_barrier()
        pltpu.sync_copy(acc.at[pl.ds(s * RPT, RPT)],
                        agg_out.at[pl.ds(s * RPT, RPT), pl.ds(qoff, QD)])
        if h == 0:
            lax.fori_loop(0, CB * (QD // L), zero_body, 0)


def _dinv_body(deg_ref, out_ref):
    deg = jnp.sum(deg_ref[...], axis=0, keepdims=True) + 1.0
    out_ref[...] = lax.rsqrt(deg)


def _scale_mm_body(x_ref, w_ref, dinv_ref, out_ref):
    h = jnp.dot(x_ref[...], w_ref[...], preferred_element_type=jnp.float32)
    out_ref[...] = h * dinv_ref[...]


def _mid_body(agg_ref, hp_ref, dinv_ref, b_ref, g_ref, bt_ref, w_ref, out_ref):
    t = (agg_ref[:N, :] + hp_ref[...]) * dinv_ref[...] + b_ref[...]
    t = t * g_ref[...] + bt_ref[...]
    t = jnp.maximum(t, 0.0)
    h2 = jnp.dot(t, w_ref[...], preferred_element_type=jnp.float32)
    out_ref[...] = h2 * dinv_ref[...]


def _final_body(agg_ref, hp_ref, dinv_ref, b_ref, out_ref):
    out_ref[...] = (agg_ref[:N, :] + hp_ref[...]) * dinv_ref[...] + b_ref[...]


def kernel(x, edge_index, W1, b1, bn_gamma, bn_beta, W2, b2):
    src = edge_index[0].reshape(NS, EPT)
    dst = edge_index[1].reshape(NS, EPT)
    srcr = jnp.pad(src, ((0, 0), (0, EPP - EPT))).reshape(NS, CH, CB)
    dstr = jnp.pad(dst, ((0, 0), (0, EPP - EPT)),
                   constant_values=DUMMY).reshape(NS, CH, CB)

    deg_parts = _deg_kernel(dstr.reshape(NW, DCH, CB))

    dinv_row = pl.pallas_call(
        _dinv_body,
        out_shape=jax.ShapeDtypeStruct((1, ACC_ROWS), jnp.float32),
    )(deg_parts)
    dinv_col = dinv_row.reshape(ACC_ROWS, 1)[:N]

    h1p = pl.pallas_call(
        _scale_mm_body,
        out_shape=jax.ShapeDtypeStruct((N, D), jnp.float32),
    )(x, W1, dinv_col)

    agg1 = _mp_kernel(h1p, srcr, dstr)

    bn_scale = (bn_gamma / jnp.sqrt(1.0 + 1e-5)).reshape(1, D)
    h2p = pl.pallas_call(
        _mid_body,
        out_shape=jax.ShapeDtypeStruct((N, D), jnp.float32),
    )(agg1, h1p, dinv_col, b1.reshape(1, D), bn_scale,
      bn_beta.reshape(1, D), W2)

    agg2 = _mp_kernel(h2p, srcr, dstr)

    out = pl.pallas_call(
        _final_body,
        out_shape=jax.ShapeDtypeStruct((N, D), jnp.float32),
    )(agg2, h2p, dinv_col, b2.reshape(1, D))
    return out


# speedup vs baseline: 1.1503x; 1.1503x over previous
import functools

import jax
import jax.numpy as jnp
from jax import lax
from jax.experimental import pallas as pl
from jax.experimental.pallas import tpu as pltpu
from jax.experimental.pallas import tpu_sc as plsc

N = 10000
D = 128
QD = D // 4
E = 320000
NC = 2
NS = 16
NW = NC * NS
L = 16

CB = 128
CH = 160
EPT = E // NS
EPP = CH * CB
DCH = (NS * EPP) // (NW * CB)
ACC_ROWS = 10112
RPT = ACC_ROWS // NS
NPT = N // NS
DUMMY = 10016

_mesh = plsc.VectorSubcoreMesh(
    core_axis_name="c", subcore_axis_name="s", num_cores=NC, num_subcores=NS)
_sc_params = pltpu.CompilerParams(
    needs_layout_passes=False, use_tc_tiling_on_sc=False)


@functools.partial(
    pl.kernel,
    out_type=jax.ShapeDtypeStruct((NW, ACC_ROWS), jnp.float32),
    mesh=_mesh,
    compiler_params=_sc_params,
    scratch_types=[
        pltpu.VMEM((DCH, CB), jnp.int32),
        pltpu.VMEM((ACC_ROWS,), jnp.float32),
    ],
)
def _deg_kernel(dstr_hbm, deg_out, dst_v, deg_v):
    wid = lax.axis_index("s") * NC + lax.axis_index("c")
    pltpu.sync_copy(dstr_hbm.at[wid], dst_v)

    zeros16 = jnp.zeros((L,), jnp.float32)

    def zero_body(i, _):
        deg_v[pl.ds(i * L, L)] = zeros16
        return 0

    lax.fori_loop(0, ACC_ROWS // L, zero_body, 0)

    ones16 = jnp.full((L,), 1.0, jnp.float32)

    def acc_body(t, _):
        j = t >> 3
        k = t & 7
        idx = dst_v[j, pl.ds(k * L, L)]
        plsc.addupdate_scatter(deg_v, [idx], ones16)
        return 0

    lax.fori_loop(0, DCH * (CB // L), acc_body, 0)
    pltpu.sync_copy(deg_v, deg_out.at[wid])


@functools.partial(
    pl.kernel,
    out_type=jax.ShapeDtypeStruct((ACC_ROWS, D), jnp.float32),
    mesh=_mesh,
    compiler_params=_sc_params,
    scratch_types=[
        pltpu.VMEM((CH, CB), jnp.int32),
        pltpu.VMEM((CH, CB), jnp.int32),
        [pltpu.VMEM((CB, QD), jnp.float32) for _ in range(4)],
        pltpu.VMEM_SHARED((N, QD), jnp.float32),
        pltpu.VMEM_SHARED((ACC_ROWS, QD), jnp.float32),
        [pltpu.SemaphoreType.DMA for _ in range(4)],
    ],
)
def _mp_kernel(hp_hbm, srcr_hbm, dstr_hbm, agg_out,
               src_v, dst_v, bufs, hp_s, acc, gsems):
    c = lax.axis_index("c")
    s = lax.axis_index("s")

    pltpu.sync_copy(srcr_hbm.at[s], src_v)
    pltpu.sync_copy(dstr_hbm.at[s], dst_v)

    r0 = bufs[0]
    zeros16 = jnp.zeros((L,), jnp.float32)

    def zero_body(t, _):
        j = t >> 1
        k = t & 1
        r0[j, pl.ds(k * L, L)] = zeros16
        return 0

    lax.fori_loop(0, CB * (QD // L), zero_body, 0)

    for h in range(2):
        for q in range(4):
            pltpu.sync_copy(r0, acc.at[pl.ds(s * RPT + q * CB, CB)])
        pltpu.sync_copy(r0.at[pl.ds(0, RPT - 4 * CB)],
                        acc.at[pl.ds(s * RPT + 4 * CB, RPT - 4 * CB)])
        qoff = (2 * c + h) * QD
        pltpu.sync_copy(hp_hbm.at[pl.ds(s * NPT, NPT), pl.ds(qoff, QD)],
                        hp_s.at[pl.ds(s * NPT, NPT)])
        plsc.subcore_barrier()

        for b in range(4):
            pltpu.async_copy(hp_s.at[src_v.at[b]], bufs[b], gsems[b])

        def body(t, _):
            j = 4 * t
            for b in range(4):
                pltpu.make_async_copy(
                    hp_s.at[src_v.at[j + b]], bufs[b], gsems[b]).wait()
                pltpu.sync_copy(bufs[b], acc.at[dst_v.at[j + b]], add=True)
                n = jnp.minimum(j + 4 + b, CH - 4 + b)
                pltpu.async_copy(hp_s.at[src_v.at[n]], bufs[b], gsems[b])
            return 0

        lax.fori_loop(0, CH // 4, body, 0)

        for b in range(4):
            pltpu.make_async_copy(
                hp_s.at[src_v.at[CH - 4 + b]], bufs[b], gsems[b]).wait()

        plsc.subcore_barrier()
        pltpu.sync_copy(acc.at[pl.ds(s * RPT, RPT)],
                        agg_out.at[pl.ds(s * RPT, RPT), pl.ds(qoff, QD)])
        if h == 0:
            lax.fori_loop(0, CB * (QD // L), zero_body, 0)


def _dinv_body(deg_ref, out_ref):
    deg = jnp.sum(deg_ref[...], axis=0, keepdims=True) + 1.0
    out_ref[...] = lax.rsqrt(deg)


def _scale_mm_body(x_ref, w_ref, dinv_ref, out_ref):
    h = jnp.dot(x_ref[...], w_ref[...], preferred_element_type=jnp.float32)
    out_ref[...] = h * dinv_ref[...]


def _mid_body(agg_ref, hp_ref, dinv_ref, b_ref, g_ref, bt_ref, w_ref, out_ref):
    t = (agg_ref[:N, :] + hp_ref[...]) * dinv_ref[...] + b_ref[...]
    t = t * g_ref[...] + bt_ref[...]
    t = jnp.maximum(t, 0.0)
    h2 = jnp.dot(t, w_ref[...], preferred_element_type=jnp.float32)
    out_ref[...] = h2 * dinv_ref[...]


def _final_body(agg_ref, hp_ref, dinv_ref, b_ref, out_ref):
    out_ref[...] = (agg_ref[:N, :] + hp_ref[...]) * dinv_ref[...] + b_ref[...]


def kernel(x, edge_index, W1, b1, bn_gamma, bn_beta, W2, b2):
    src = edge_index[0].reshape(NS, EPT)
    dst = edge_index[1].reshape(NS, EPT)
    srcr = jnp.pad(src, ((0, 0), (0, EPP - EPT))).reshape(NS, CH, CB)
    dstr = jnp.pad(dst, ((0, 0), (0, EPP - EPT)),
                   constant_values=DUMMY).reshape(NS, CH, CB)

    deg_parts = _deg_kernel(dstr.reshape(NW, DCH, CB))

    dinv_row = pl.pallas_call(
        _dinv_body,
        out_shape=jax.ShapeDtypeStruct((1, ACC_ROWS), jnp.float32),
    )(deg_parts)
    dinv_col = dinv_row.reshape(ACC_ROWS, 1)[:N]

    h1p = pl.pallas_call(
        _scale_mm_body,
        out_shape=jax.ShapeDtypeStruct((N, D), jnp.float32),
    )(x, W1, dinv_col)

    agg1 = _mp_kernel(h1p, srcr, dstr)

    bn_scale = (bn_gamma / jnp.sqrt(1.0 + 1e-5)).reshape(1, D)
    h2p = pl.pallas_call(
        _mid_body,
        out_shape=jax.ShapeDtypeStruct((N, D), jnp.float32),
    )(agg1, h1p, dinv_col, b1.reshape(1, D), bn_scale,
      bn_beta.reshape(1, D), W2)

    agg2 = _mp_kernel(h2p, srcr, dstr)

    out = pl.pallas_call(
        _final_body,
        out_shape=jax.ShapeDtypeStruct((N, D), jnp.float32),
    )(agg2, h2p, dinv_col, b2.reshape(1, D))
    return out


# speedup vs baseline: 1.1741x; 1.0207x over previous
import functools

import jax
import jax.numpy as jnp
from jax import lax
from jax.experimental import pallas as pl
from jax.experimental.pallas import tpu as pltpu
from jax.experimental.pallas import tpu_sc as plsc

N = 10000
D = 128
QD = D // 4
E = 320000
NC = 2
NS = 16
NW = NC * NS
L = 16

CB = 128
CH = 160
EPT = E // NS
EPP = CH * CB
DCH = (NS * EPP) // (NW * CB)
ACC_ROWS = 10112
RPT = ACC_ROWS // NS
NPT = N // NS
DUMMY = 10016

_mesh = plsc.VectorSubcoreMesh(
    core_axis_name="c", subcore_axis_name="s", num_cores=NC, num_subcores=NS)
_sc_params = pltpu.CompilerParams(
    needs_layout_passes=False, use_tc_tiling_on_sc=False)


@functools.partial(
    pl.kernel,
    out_type=jax.ShapeDtypeStruct((NW, ACC_ROWS), jnp.float32),
    mesh=_mesh,
    compiler_params=_sc_params,
    scratch_types=[
        pltpu.VMEM((DCH, CB), jnp.int32),
        pltpu.VMEM((ACC_ROWS,), jnp.float32),
    ],
)
def _deg_kernel(dstr_hbm, deg_out, dst_v, deg_v):
    wid = lax.axis_index("s") * NC + lax.axis_index("c")
    pltpu.sync_copy(dstr_hbm.at[wid], dst_v)

    zeros16 = jnp.zeros((L,), jnp.float32)

    def zero_body(i, _):
        deg_v[pl.ds(i * L, L)] = zeros16
        return 0

    lax.fori_loop(0, ACC_ROWS // L, zero_body, 0)

    ones16 = jnp.full((L,), 1.0, jnp.float32)

    def acc_body(t, _):
        j = t >> 3
        k = t & 7
        idx = dst_v[j, pl.ds(k * L, L)]
        plsc.addupdate_scatter(deg_v, [idx], ones16)
        return 0

    lax.fori_loop(0, DCH * (CB // L), acc_body, 0)
    pltpu.sync_copy(deg_v, deg_out.at[wid])


@functools.partial(
    pl.kernel,
    out_type=jax.ShapeDtypeStruct((ACC_ROWS, D), jnp.float32),
    mesh=_mesh,
    compiler_params=_sc_params,
    scratch_types=[
        pltpu.VMEM((CH, CB), jnp.int32),
        pltpu.VMEM((CH, CB), jnp.int32),
        [pltpu.VMEM((CB, QD), jnp.float32) for _ in range(2)],
        pltpu.VMEM((CB, QD), jnp.float32),
        pltpu.VMEM_SHARED((N, QD), jnp.float32),
        [pltpu.VMEM_SHARED((ACC_ROWS, QD), jnp.float32)
         for _ in range(2)],
        [pltpu.SemaphoreType.DMA for _ in range(2)],
        pltpu.SemaphoreType.DMA,
    ],
)
def _mp_kernel(hp_hbm, srcr_hbm, dstr_hbm, agg_out,
               src_v, dst_v, bufs, zbuf, hp_s, accs, gsems, dsem):
    c = lax.axis_index("c")
    s = lax.axis_index("s")

    r0, r1 = bufs
    sem0, sem1 = gsems
    idx0 = pltpu.async_copy(srcr_hbm.at[s], src_v, sem0)
    idx1 = pltpu.async_copy(dstr_hbm.at[s], dst_v, sem1)
    zeros16 = jnp.zeros((L,), jnp.float32)

    def zero_body(t, _):
        j = t >> 1
        k = t & 1
        zbuf[j, pl.ds(k * L, L)] = zeros16
        return 0

    lax.fori_loop(0, CB * (QD // L), zero_body, 0)
    idx0.wait()
    idx1.wait()

    for h in range(2):
        acc = accs[h]
        for q in range(4):
            pltpu.sync_copy(zbuf, acc.at[pl.ds(s * RPT + q * CB, CB)])
        pltpu.sync_copy(zbuf.at[pl.ds(0, RPT - 4 * CB)],
                        acc.at[pl.ds(s * RPT + 4 * CB, RPT - 4 * CB)])
        qoff = (2 * c + h) * QD
        pltpu.sync_copy(hp_hbm.at[pl.ds(s * NPT, NPT), pl.ds(qoff, QD)],
                        hp_s.at[pl.ds(s * NPT, NPT)])
        plsc.subcore_barrier()

        pltpu.async_copy(hp_s.at[src_v.at[0]], r0, sem0)
        pltpu.async_copy(hp_s.at[src_v.at[1]], r1, sem1)

        def body(t, _):
            j0 = 2 * t
            j1 = j0 + 1
            pltpu.make_async_copy(hp_s.at[src_v.at[j0]], r0, sem0).wait()
            pltpu.sync_copy(r0, acc.at[dst_v.at[j0]], add=True)
            n0 = jnp.minimum(j0 + 2, CH - 2)
            pltpu.async_copy(hp_s.at[src_v.at[n0]], r0, sem0)
            pltpu.make_async_copy(hp_s.at[src_v.at[j1]], r1, sem1).wait()
            pltpu.sync_copy(r1, acc.at[dst_v.at[j1]], add=True)
            n1 = jnp.minimum(j1 + 2, CH - 1)
            pltpu.async_copy(hp_s.at[src_v.at[n1]], r1, sem1)
            return 0

        lax.fori_loop(0, CH // 2, body, 0)

        pltpu.make_async_copy(hp_s.at[src_v.at[CH - 2]], r0, sem0).wait()
        pltpu.make_async_copy(hp_s.at[src_v.at[CH - 1]], r1, sem1).wait()

        plsc.subcore_barrier()
        dump = pltpu.async_copy(
            acc.at[pl.ds(s * RPT, RPT)],
            agg_out.at[pl.ds(s * RPT, RPT), pl.ds(qoff, QD)], dsem)
        if h == 1:
            dump.wait()

    pltpu.make_async_copy(
        accs[0].at[pl.ds(s * RPT, RPT)],
        agg_out.at[pl.ds(s * RPT, RPT), pl.ds(0 * QD, QD)], dsem).wait()


def _dinv_body(deg_ref, out_ref):
    deg = jnp.sum(deg_ref[...], axis=0, keepdims=True) + 1.0
    out_ref[...] = lax.rsqrt(deg)


def _scale_mm_body(x_ref, w_ref, dinv_ref, out_ref):
    h = jnp.dot(x_ref[...], w_ref[...], preferred_element_type=jnp.float32)
    out_ref[...] = h * dinv_ref[...]


def _mid_body(agg_ref, hp_ref, dinv_ref, b_ref, g_ref, bt_ref, w_ref, out_ref):
    t = (agg_ref[:N, :] + hp_ref[...]) * dinv_ref[...] + b_ref[...]
    t = t * g_ref[...] + bt_ref[...]
    t = jnp.maximum(t, 0.0)
    h2 = jnp.dot(t, w_ref[...], preferred_element_type=jnp.float32)
    out_ref[...] = h2 * dinv_ref[...]


def _final_body(agg_ref, hp_ref, dinv_ref, b_ref, out_ref):
    out_ref[...] = (agg_ref[:N, :] + hp_ref[...]) * dinv_ref[...] + b_ref[...]


def kernel(x, edge_index, W1, b1, bn_gamma, bn_beta, W2, b2):
    src = edge_index[0].reshape(NS, EPT)
    dst = edge_index[1].reshape(NS, EPT)
    srcr = jnp.pad(src, ((0, 0), (0, EPP - EPT))).reshape(NS, CH, CB)
    dstr = jnp.pad(dst, ((0, 0), (0, EPP - EPT)),
                   constant_values=DUMMY).reshape(NS, CH, CB)

    deg_parts = _deg_kernel(dstr.reshape(NW, DCH, CB))

    dinv_row = pl.pallas_call(
        _dinv_body,
        out_shape=jax.ShapeDtypeStruct((1, ACC_ROWS), jnp.float32),
    )(deg_parts)
    dinv_col = dinv_row.reshape(ACC_ROWS, 1)[:N]

    h1p = pl.pallas_call(
        _scale_mm_body,
        out_shape=jax.ShapeDtypeStruct((N, D), jnp.float32),
    )(x, W1, dinv_col)

    agg1 = _mp_kernel(h1p, srcr, dstr)

    bn_scale = (bn_gamma / jnp.sqrt(1.0 + 1e-5)).reshape(1, D)
    h2p = pl.pallas_call(
        _mid_body,
        out_shape=jax.ShapeDtypeStruct((N, D), jnp.float32),
    )(agg1, h1p, dinv_col, b1.reshape(1, D), bn_scale,
      bn_beta.reshape(1, D), W2)

    agg2 = _mp_kernel(h2p, srcr, dstr)

    out = pl.pallas_call(
        _final_body,
        out_shape=jax.ShapeDtypeStruct((N, D), jnp.float32),
    )(agg2, h2p, dinv_col, b2.reshape(1, D))
    return out
